# baseline (speedup 1.0000x reference)
import dataclasses
import functools

import jax
import jax.numpy as jnp
from jax import lax
from jax.experimental import pallas as pl
from jax.experimental.pallas import tpu as pltpu
from jax.experimental.pallas import tpu_sc as plsc

_D = 4
_TW = 6
_PW = 4
_EPS = 1e-5
_L = 16
_NC, _NS = 2, 16
_NW = _NC * _NS


def _tables_body(x_ref, e_ref, wx_ref, we_ref, tx_ref, te_ref):
    df = wx_ref.shape[0]
    pid = pl.program_id(0)

    def one(src, w, dst):
        m = (src[0::_D, :] + src[1::_D, :] + src[2::_D, :]
             + src[3::_D, :]) * 0.25
        a = jnp.dot(m, w[...], preferred_element_type=jnp.float32,
                    precision=lax.Precision.HIGHEST)
        q = jnp.sum(m * m, axis=1, keepdims=True)
        colid = lax.broadcasted_iota(jnp.int32, a.shape, 1)
        bn = a.shape[0]
        t = a + jnp.where(colid == _TW - 1, q, 0.0)
        u = lax.bitcast_convert_type(t[:, :4], jnp.int32)
        ur = u + (0x7FFF + (lax.shift_right_logical(u, 16) & 1))
        top = ur & jnp.int32(-65536)
        w01 = lax.shift_right_logical(top[:, 0:1], 16) | top[:, 1:2]
        w23 = lax.shift_right_logical(top[:, 2:3], 16) | top[:, 3:4]
        sq = lax.bitcast_convert_type(t[:, 4:6], jnp.int32)
        dst[pl.ds(pid * bn, bn), :] = jnp.concatenate([w01, w23, sq], axis=1)

    one(x_ref, wx_ref, tx_ref)
    one(e_ref, we_ref, te_ref)


def _build_tables(x, e, wx, we):
    nn = x.shape[0] // _D
    ne = e.shape[0] // _D
    df = wx.shape[0]
    bn = 2000
    grid = (nn // bn,)
    return pl.pallas_call(
        _tables_body,
        grid=grid,
        in_specs=[
            pl.BlockSpec((bn * _D, df), lambda i: (i, 0)),
            pl.BlockSpec((bn * _D, df), lambda i: (i, 0)),
            pl.BlockSpec((df, _TW), lambda i: (0, 0)),
            pl.BlockSpec((df, _TW), lambda i: (0, 0)),
        ],
        out_specs=[
            pl.BlockSpec((nn, _PW), lambda i: (0, 0)),
            pl.BlockSpec((ne, _PW), lambda i: (0, 0)),
        ],
        out_shape=[
            jax.ShapeDtypeStruct((nn, _PW), jnp.int32),
            jax.ShapeDtypeStruct((ne, _PW), jnp.int32),
        ],
    )(x, e, wx, we)


def _sc_combine(tx, te, hi, c0):
    nn = tx.size // _PW
    ne = te.size // _PW
    E = hi.shape[1]
    C = 640
    txf = tx.reshape(nn * _PW)
    tef = te.reshape(ne * _PW)
    c0p = jnp.concatenate([c0, jnp.zeros((4,), jnp.float32)])
    mesh = plsc.VectorSubcoreMesh(core_axis_name="c", subcore_axis_name="s")
    cp = pltpu.CompilerParams()
    if "needs_layout_passes" in pltpu.CompilerParams.__dataclass_fields__:
        cp = dataclasses.replace(cp, needs_layout_passes=False)

    @functools.partial(
        pl.kernel,
        compiler_params=cp,
        out_type=jax.ShapeDtypeStruct((_D, E), jnp.float32),
        mesh=mesh,
        scratch_types=[
            pltpu.VMEM((nn * _PW,), jnp.int32),
            pltpu.VMEM((ne * _PW,), jnp.int32),
            pltpu.VMEM((8,), jnp.float32),
            pltpu.SemaphoreType.DMA,
            pltpu.SemaphoreType.DMA,
        ],
    )
    def sc_kernel(tx_hbm, te_hbm, hi1_hbm, hi2_hbm, c0_hbm, out_hbm,
                  txv, tev, c0v, sem1, sem2):
        cp1 = pltpu.async_copy(tx_hbm, txv, sem1)
        cp2 = pltpu.async_copy(te_hbm, tev, sem2)
        pltpu.sync_copy(c0_hbm, c0v)
        cp1.wait()
        cp2.wait()

        def body(rowv, colv, outv):
            @plsc.parallel_loop(0, C, step=_L, unroll=8)
            def _vec(j):
                rb = rowv[0, pl.ds(j, _L)] * _PW
                cb = colv[0, pl.ds(j, _L)] * _PW

                def g(tab, ids, k):
                    return plsc.load_gather(tab, [ids + k])

                def unpack(w):
                    lo = plsc.bitcast(w << 16, jnp.float32)
                    hi_ = plsc.bitcast(w & jnp.int32(-65536), jnp.float32)
                    return lo, hi_

                x01, x23 = g(txv, rb, 0), g(txv, rb, 1)
                e01, e23 = g(tev, cb, 0), g(tev, cb, 1)
                px0, px1 = unpack(x01)
                px2, px3 = unpack(x23)
                pe0, pe1 = unpack(e01)
                pe2, pe3 = unpack(e23)
                p = [px0 + pe0, px1 + pe1, px2 + pe2, px3 + pe3]
                ssum = (plsc.bitcast(g(txv, rb, 2), jnp.float32)
                        + plsc.bitcast(g(tev, cb, 2), jnp.float32))
                qsum = (plsc.bitcast(g(txv, rb, 3), jnp.float32)
                        + plsc.bitcast(g(tev, cb, 3), jnp.float32))
                mu = ssum * (1.0 / 256.0)
                v = qsum * (1.0 / 256.0) - mu * mu + _EPS
                iv = plsc.bitcast(v, jnp.int32)
                y = plsc.bitcast(jnp.int32(0x5F3759DF) - (iv >> 1), jnp.float32)
                y = y * (1.5 - 0.5 * v * y * y)
                y = y * (1.5 - 0.5 * v * y * y)
                for k in range(_D):
                    c0k = plsc.load_gather(
                        c0v, [jnp.full((_L,), k, jnp.int32)])
                    z = p[k] * y + c0k
                    o = 1.0 / (1.0 + jnp.exp(-z))
                    outv[k, pl.ds(j, _L)] = o

        pltpu.emit_pipeline(
            body,
            grid=(E // C,),
            in_specs=[pl.BlockSpec((1, C), lambda i: (0, i)),
                      pl.BlockSpec((1, C), lambda i: (1, i))],
            out_specs=[pl.BlockSpec((_D, C), lambda i: (0, i))],
            core_axis_name=("c", "s"),
            dimension_semantics=(pltpu.PARALLEL,),
        )(hi1_hbm, hi2_hbm, out_hbm)

    return sc_kernel(txf, tef, hi, hi, c0p).T


def kernel(x, e, hyperedge_index, node_types, hyperedge_types,
           ln_gamma, ln_beta, W, b):
    df = x.shape[-1]
    Wg = ln_gamma[:, None] * W
    gw = ln_gamma @ W
    c0 = ln_beta @ W + b
    ones = jnp.ones((df, 1), jnp.float32)
    zeros = jnp.zeros((df, 1), jnp.float32)
    wx = jnp.concatenate([Wg[:df] - gw[None, :] * (1.0 / 256.0), ones, zeros],
                         axis=1)
    we = jnp.concatenate([Wg[df:] - gw[None, :] * (1.0 / 256.0), ones, zeros],
                         axis=1)
    hi = hyperedge_index.astype(jnp.int32)
    tx, te = _build_tables(x, e, wx, we)
    return _sc_combine(tx, te, hi, c0)


# speedup vs baseline: 1.0022x; 1.0022x over previous
import dataclasses
import functools

import jax
import jax.numpy as jnp
from jax import lax
from jax.experimental import pallas as pl
from jax.experimental.pallas import tpu as pltpu
from jax.experimental.pallas import tpu_sc as plsc

_D = 4
_TW = 6
_PW = 4
_EPS = 1e-5
_L = 16
_NC, _NS = 2, 16
_NW = _NC * _NS


def _tables_body(x_ref, e_ref, wx_ref, we_ref, tx_ref, te_ref):
    df = wx_ref.shape[0]
    pid = pl.program_id(0)

    def one(src, w, dst):
        m = (src[0::_D, :] + src[1::_D, :] + src[2::_D, :]
             + src[3::_D, :]) * 0.25
        a = jnp.dot(m, w[...], preferred_element_type=jnp.float32,
                    precision=lax.Precision.HIGHEST)
        q = jnp.sum(m * m, axis=1, keepdims=True)
        colid = lax.broadcasted_iota(jnp.int32, a.shape, 1)
        bn = a.shape[0]
        t = a + jnp.where(colid == _TW - 1, q, 0.0)
        u = lax.bitcast_convert_type(t[:, :4], jnp.int32)
        ur = u + (0x7FFF + (lax.shift_right_logical(u, 16) & 1))
        top = ur & jnp.int32(-65536)
        w01 = lax.shift_right_logical(top[:, 0:1], 16) | top[:, 1:2]
        w23 = lax.shift_right_logical(top[:, 2:3], 16) | top[:, 3:4]
        sq = lax.bitcast_convert_type(t[:, 4:6], jnp.int32)
        dst[pl.ds(pid * bn, bn), :] = jnp.concatenate([w01, w23, sq], axis=1)

    one(x_ref, wx_ref, tx_ref)
    one(e_ref, we_ref, te_ref)


def _build_tables(x, e, wx, we):
    nn = x.shape[0] // _D
    ne = e.shape[0] // _D
    df = wx.shape[0]
    bn = 2000
    grid = (nn // bn,)
    return pl.pallas_call(
        _tables_body,
        grid=grid,
        in_specs=[
            pl.BlockSpec((bn * _D, df), lambda i: (i, 0)),
            pl.BlockSpec((bn * _D, df), lambda i: (i, 0)),
            pl.BlockSpec((df, _TW), lambda i: (0, 0)),
            pl.BlockSpec((df, _TW), lambda i: (0, 0)),
        ],
        out_specs=[
            pl.BlockSpec((nn, _PW), lambda i: (0, 0)),
            pl.BlockSpec((ne, _PW), lambda i: (0, 0)),
        ],
        out_shape=[
            jax.ShapeDtypeStruct((nn, _PW), jnp.int32),
            jax.ShapeDtypeStruct((ne, _PW), jnp.int32),
        ],
    )(x, e, wx, we)


def _sc_combine(tx, te, hi, c0):
    nn = tx.size // _PW
    ne = te.size // _PW
    E = hi.shape[1]
    C = 1280
    txf = tx.reshape(nn * _PW)
    tef = te.reshape(ne * _PW)
    c0p = jnp.concatenate([c0, jnp.zeros((4,), jnp.float32)])
    mesh = plsc.VectorSubcoreMesh(core_axis_name="c", subcore_axis_name="s")
    cp = pltpu.CompilerParams()
    if "needs_layout_passes" in pltpu.CompilerParams.__dataclass_fields__:
        cp = dataclasses.replace(cp, needs_layout_passes=False)

    @functools.partial(
        pl.kernel,
        compiler_params=cp,
        out_type=jax.ShapeDtypeStruct((_D, E), jnp.float32),
        mesh=mesh,
        scratch_types=[
            pltpu.VMEM((nn * _PW,), jnp.int32),
            pltpu.VMEM((ne * _PW,), jnp.int32),
            pltpu.VMEM((8,), jnp.float32),
            pltpu.SemaphoreType.DMA,
            pltpu.SemaphoreType.DMA,
        ],
    )
    def sc_kernel(tx_hbm, te_hbm, hi1_hbm, hi2_hbm, c0_hbm, out_hbm,
                  txv, tev, c0v, sem1, sem2):
        cp1 = pltpu.async_copy(tx_hbm, txv, sem1)
        cp2 = pltpu.async_copy(te_hbm, tev, sem2)
        pltpu.sync_copy(c0_hbm, c0v)
        cp1.wait()
        cp2.wait()

        def body(rowv, colv, outv):
            @plsc.parallel_loop(0, C, step=_L, unroll=8)
            def _vec(j):
                rb = rowv[0, pl.ds(j, _L)] * _PW
                cb = colv[0, pl.ds(j, _L)] * _PW

                def g(tab, ids, k):
                    return plsc.load_gather(tab, [ids + k])

                def unpack(w):
                    lo = plsc.bitcast(w << 16, jnp.float32)
                    hi_ = plsc.bitcast(w & jnp.int32(-65536), jnp.float32)
                    return lo, hi_

                x01, x23 = g(txv, rb, 0), g(txv, rb, 1)
                e01, e23 = g(tev, cb, 0), g(tev, cb, 1)
                px0, px1 = unpack(x01)
                px2, px3 = unpack(x23)
                pe0, pe1 = unpack(e01)
                pe2, pe3 = unpack(e23)
                p = [px0 + pe0, px1 + pe1, px2 + pe2, px3 + pe3]
                ssum = (plsc.bitcast(g(txv, rb, 2), jnp.float32)
                        + plsc.bitcast(g(tev, cb, 2), jnp.float32))
                qsum = (plsc.bitcast(g(txv, rb, 3), jnp.float32)
                        + plsc.bitcast(g(tev, cb, 3), jnp.float32))
                mu = ssum * (1.0 / 256.0)
                v = qsum * (1.0 / 256.0) - mu * mu + _EPS
                iv = plsc.bitcast(v, jnp.int32)
                y = plsc.bitcast(jnp.int32(0x5F3759DF) - (iv >> 1), jnp.float32)
                y = y * (1.5 - 0.5 * v * y * y)
                y = y * (1.5 - 0.5 * v * y * y)
                for k in range(_D):
                    c0k = plsc.load_gather(
                        c0v, [jnp.full((_L,), k, jnp.int32)])
                    z = p[k] * y + c0k
                    o = 1.0 / (1.0 + jnp.exp(-z))
                    outv[k, pl.ds(j, _L)] = o

        pltpu.emit_pipeline(
            body,
            grid=(E // C,),
            in_specs=[pl.BlockSpec((1, C), lambda i: (0, i)),
                      pl.BlockSpec((1, C), lambda i: (1, i))],
            out_specs=[pl.BlockSpec((_D, C), lambda i: (0, i))],
            core_axis_name=("c", "s"),
            dimension_semantics=(pltpu.PARALLEL,),
        )(hi1_hbm, hi2_hbm, out_hbm)

    return sc_kernel(txf, tef, hi, hi, c0p).T


def kernel(x, e, hyperedge_index, node_types, hyperedge_types,
           ln_gamma, ln_beta, W, b):
    df = x.shape[-1]
    Wg = ln_gamma[:, None] * W
    gw = ln_gamma @ W
    c0 = ln_beta @ W + b
    ones = jnp.ones((df, 1), jnp.float32)
    zeros = jnp.zeros((df, 1), jnp.float32)
    wx = jnp.concatenate([Wg[:df] - gw[None, :] * (1.0 / 256.0), ones, zeros],
                         axis=1)
    we = jnp.concatenate([Wg[df:] - gw[None, :] * (1.0 / 256.0), ones, zeros],
                         axis=1)
    hi = hyperedge_index.astype(jnp.int32)
    tx, te = _build_tables(x, e, wx, we)
    return _sc_combine(tx, te, hi, c0)
